# baseline (speedup 1.0000x reference)
import jax
import jax.numpy as jnp
from jax import lax
from jax.experimental import pallas as pl
from jax.experimental.pallas import tpu as pltpu
from jax.experimental.pallas import tpu_sc as plsc

NUM_EMBEDDINGS = 1000000
EMBED_DIM = 32
BATCH = 4096
SEQ = 200
TOTAL = BATCH * SEQ

NC = 2
NS = 16
NW = NC * NS
LANES = 16

UNIT = 128
N_UNITS = TOTAL // UNIT
U_PER_W = N_UNITS // NW
N_BLOCKS = N_UNITS // 8
B_PER_W = N_BLOCKS // NW
CBLK = EMBED_DIM // 8
BBLK = BATCH // UNIT
RING = 4
OUTB = 2


def _body(table_hbm, idx_hbm, out_hbm, idx_v, rows_v, t_v, gsem, osem):
  wid = lax.axis_index("s") * NC + lax.axis_index("c")
  blk0 = wid * B_PER_W

  pltpu.sync_copy(idx_hbm.at[pl.ds(blk0, B_PER_W)], idx_v)

  lane = lax.iota(jnp.int32, LANES)
  diag_cols = [[((lane + d) % LANES) + h * LANES for d in range(LANES)]
               for h in range(2)]

  def idx_row(k):
    return idx_v.at[k // 8, k % 8]

  def fire_gather(k, r):
    pltpu.async_copy(table_hbm.at[idx_row(k)], rows_v.at[r], gsem.at[r])

  def drain(sem_ref, vmem_ref, hbm_like):
    pltpu.make_async_copy(hbm_like, vmem_ref, sem_ref).wait()

  for r in range(RING - 1):
    fire_gather(r, r)

  @pl.loop(0, U_PER_W, step=RING)
  def _ring(kstart):
    for r in range(RING):
      k = kstart + r
      b_id = blk0 + k // 8
      s = (b_id // BBLK) * 8 + k % 8
      bt = b_id % BBLK
      p = r % OUTB

      @pl.when(k + RING - 1 < U_PER_W)
      def _fire_next():
        fire_gather(k + RING - 1, (r + RING - 1) % RING)

      drain(gsem.at[r], rows_v.at[r], table_hbm.at[pl.ds(0, UNIT)])

      @pl.when(k >= OUTB)
      def _drain_prev_out():
        drain(osem.at[p], t_v.at[p], out_hbm.at[0, :, 0])

      @pl.loop(0, UNIT // LANES, unroll=4)
      def _g(g):
        vidx = idx_row(k)[pl.ds(g * LANES, LANES)]
        keep = vidx != 0
        row_ids = lane + g * LANES
        for h in range(2):
          for d in range(LANES):
            cols = diag_cols[h][d]
            v = plsc.load_gather(rows_v.at[r], [row_ids, cols])
            v = jnp.where(keep, v, 0.0)
            plsc.store_scatter(t_v.at[p], [cols // 8, cols % 8, row_ids], v)

      pltpu.async_copy(t_v.at[p], out_hbm.at[s, :, bt], osem.at[p])

  for p in range(OUTB):
    drain(osem.at[p], t_v.at[p], out_hbm.at[0, :, 0])


N_TC = 7813
PAD_ROWS = N_TC * UNIT


def _retile_body(wt_hbm, tail_hbm, out_hbm, in_v, st_v, isem, osem):
  wid = lax.axis_index("s") * NC + lax.axis_index("c")
  lane = lax.iota(jnp.int32, LANES)
  n_my = N_TC // NW + 1

  def drain(sem_ref, vmem_ref, hbm_like):
    pltpu.make_async_copy(hbm_like, vmem_ref, sem_ref).wait()

  def fire_in(tc, b):
    pltpu.async_copy(wt_hbm.at[:, pl.ds(tc * UNIT, UNIT)], in_v.at[b],
                     isem.at[b])

  fire_in(wid, 0)

  @pl.loop(0, n_my)
  def _unit(t):
    tc = wid + t * NW

    @pl.when(tc < N_TC)
    def _do():
      b = t % 2

      @pl.when(tc + NW < N_TC - 1)
      def _fire_next():
        fire_in(tc + NW, 1 - b)

      @pl.when(tc < N_TC - 1)
      def _drain_in():
        drain(isem.at[b], in_v.at[b], out_hbm.at[pl.ds(0, EMBED_DIM)])

      @pl.when(tc == N_TC - 1)
      def _partial_in():
        pltpu.sync_copy(tail_hbm, in_v.at[b])

      @pl.when(t >= 2)
      def _drain_prev_out():
        drain(osem.at[b], st_v.at[b], out_hbm.at[pl.ds(0, EMBED_DIM)])

      for q in range(16):
        lb = q // 2
        ch = q % 2
        j_ids = lane + lb * LANES
        for d in range(LANES):
          c_ids = ((lane + d) % LANES) + ch * LANES
          v = plsc.load_gather(in_v.at[b], [c_ids, j_ids])
          plsc.store_scatter(st_v.at[b],
                             [j_ids // 4, (j_ids % 4) * EMBED_DIM + c_ids], v)

      pltpu.async_copy(st_v.at[b],
                       out_hbm.at[pl.ds(tc * EMBED_DIM, EMBED_DIM)],
                       osem.at[b])

  for b in range(2):
    drain(osem.at[b], st_v.at[b], out_hbm.at[pl.ds(0, EMBED_DIM)])


@jax.jit
def _retile(wt, wt_tail):
  mesh = plsc.VectorSubcoreMesh(core_axis_name="c", subcore_axis_name="s")
  f = pl.kernel(
      _retile_body,
      out_type=jax.ShapeDtypeStruct((N_TC * EMBED_DIM, UNIT), jnp.float32),
      mesh=mesh,
      scratch_types=[
          pltpu.VMEM((2, EMBED_DIM, UNIT), jnp.float32),
          pltpu.VMEM((2, EMBED_DIM, UNIT), jnp.float32),
          pltpu.SemaphoreType.DMA((2,)),
          pltpu.SemaphoreType.DMA((2,)),
      ],
      compiler_params=pltpu.CompilerParams(
          needs_layout_passes=False, use_tc_tiling_on_sc=True),
  )
  return f(wt, wt_tail)


@jax.jit
def _lookup(idx3d, weight):
  mesh = plsc.VectorSubcoreMesh(core_axis_name="c", subcore_axis_name="s")
  f = pl.kernel(
      _body,
      out_type=jax.ShapeDtypeStruct((SEQ, CBLK, BBLK, 8, UNIT), jnp.float32),
      mesh=mesh,
      scratch_types=[
          pltpu.VMEM((B_PER_W, 8, UNIT), jnp.int32),
          pltpu.VMEM((RING, UNIT, EMBED_DIM), jnp.float32),
          pltpu.VMEM((OUTB, CBLK, 8, UNIT), jnp.float32),
          pltpu.SemaphoreType.DMA((RING,)),
          pltpu.SemaphoreType.DMA((OUTB,)),
      ],
      compiler_params=pltpu.CompilerParams(
          needs_layout_passes=False, use_tc_tiling_on_sc=False),
  )
  return f(weight, idx3d)


def kernel(input_batch, weight):
  wt_tail = jnp.concatenate(
      [weight[NUM_EMBEDDINGS - UNIT // 2:],
       jnp.zeros((UNIT // 2, EMBED_DIM), jnp.float32)]).T
  table = _retile(weight.T, wt_tail).reshape(PAD_ROWS, EMBED_DIM)
  idx3d = (input_batch.T.reshape(SEQ // 8, 8, BBLK, UNIT)
           .transpose(0, 2, 1, 3).reshape(N_BLOCKS, 8, UNIT))
  out5 = _lookup(idx3d, table)
  return out5.transpose(2, 4, 0, 1, 3).reshape(BATCH, SEQ, EMBED_DIM)


# speedup vs baseline: 1.4138x; 1.4138x over previous
import jax
import jax.numpy as jnp
from jax import lax
from jax.experimental import pallas as pl
from jax.experimental.pallas import tpu as pltpu
from jax.experimental.pallas import tpu_sc as plsc

NUM_EMBEDDINGS = 1000000
EMBED_DIM = 32
BATCH = 4096
SEQ = 200
TOTAL = BATCH * SEQ

NC = 2
NS = 16
NW = NC * NS
LANES = 16

UNIT = 128
N_UNITS = TOTAL // UNIT
U_PER_W = N_UNITS // NW
N_BLOCKS = N_UNITS // 8
B_PER_W = N_BLOCKS // NW
CBLK = EMBED_DIM // 8
BBLK = BATCH // UNIT
RING = 6
OUTB = 4


def _body(table_hbm, idx_hbm, out_hbm, idx_v, rows_v, t_v, gsem, osem):
  wid = lax.axis_index("s") * NC + lax.axis_index("c")
  blk0 = wid * B_PER_W

  pltpu.sync_copy(idx_hbm.at[pl.ds(blk0, B_PER_W)], idx_v)

  lane = lax.iota(jnp.int32, LANES)
  diag_cols = [[((lane + d) % LANES) + h * LANES for d in range(LANES)]
               for h in range(2)]

  def idx_row(k):
    return idx_v.at[k // 8, k % 8]

  def fire_gather(k, r):
    pltpu.async_copy(table_hbm.at[idx_row(k)], rows_v.at[r], gsem.at[r])

  def drain(sem_ref, vmem_ref, hbm_like):
    pltpu.make_async_copy(hbm_like, vmem_ref, sem_ref).wait()

  for r in range(RING - 1):
    fire_gather(r, r)

  @pl.loop(0, U_PER_W)
  def _ring(k):
    r = k % RING
    p = k % OUTB
    b_id = blk0 + k // 8
    s = (b_id // BBLK) * 8 + k % 8
    bt = b_id % BBLK

    @pl.when(k + RING - 1 < U_PER_W)
    def _fire_next():
      fire_gather(k + RING - 1, (k + RING - 1) % RING)

    drain(gsem.at[r], rows_v.at[r], table_hbm.at[pl.ds(0, UNIT)])

    @pl.when(k >= OUTB)
    def _drain_prev_out():
      drain(osem.at[p], t_v.at[p], out_hbm.at[0, :, 0])

    @pl.loop(0, UNIT // LANES)
    def _g(g):
      vidx = idx_row(k)[pl.ds(g * LANES, LANES)]
      keep = vidx != 0
      row_ids = lane + g * LANES
      for h in range(2):
        for d in range(LANES):
          cols = diag_cols[h][d]
          v = plsc.load_gather(rows_v.at[r], [row_ids, cols])
          v = jnp.where(keep, v, 0.0)
          plsc.store_scatter(t_v.at[p], [cols // 8, cols % 8, row_ids], v)

    pltpu.async_copy(t_v.at[p], out_hbm.at[s, :, bt], osem.at[p])

  for p in range(OUTB):
    drain(osem.at[p], t_v.at[p], out_hbm.at[0, :, 0])


N_TC = 7813
PAD_ROWS = N_TC * UNIT


def _retile_body(wt_hbm, tail_hbm, out_hbm, in_v, st_v, isem, osem):
  wid = lax.axis_index("s") * NC + lax.axis_index("c")
  lane = lax.iota(jnp.int32, LANES)
  n_my = N_TC // NW + 1

  def drain(sem_ref, vmem_ref, hbm_like):
    pltpu.make_async_copy(hbm_like, vmem_ref, sem_ref).wait()

  def fire_in(tc, b):
    pltpu.async_copy(wt_hbm.at[:, pl.ds(tc * UNIT, UNIT)], in_v.at[b],
                     isem.at[b])

  for t0 in range(3):
    fire_in(wid + t0 * NW, t0)

  @pl.loop(0, n_my)
  def _unit(t):
    tc = wid + t * NW

    @pl.when(tc < N_TC)
    def _do():
      b = t % 4

      @pl.when(tc + 3 * NW < N_TC - 1)
      def _fire_next():
        fire_in(tc + 3 * NW, (t + 3) % 4)

      @pl.when(tc < N_TC - 1)
      def _drain_in():
        drain(isem.at[b], in_v.at[b], out_hbm.at[pl.ds(0, EMBED_DIM)])

      @pl.when(tc == N_TC - 1)
      def _partial_in():
        pltpu.sync_copy(tail_hbm, in_v.at[b])

      @pl.when(t >= 4)
      def _drain_prev_out():
        drain(osem.at[b], st_v.at[b], out_hbm.at[pl.ds(0, EMBED_DIM)])

      @pl.loop(0, 16)
      def _blk(q):
        lb = q // 2
        ch = q % 2
        j_ids = lane + lb * LANES
        for d in range(LANES):
          c_ids = ((lane + d) % LANES) + ch * LANES
          v = plsc.load_gather(in_v.at[b], [c_ids, j_ids])
          plsc.store_scatter(st_v.at[b],
                             [j_ids // 4, (j_ids % 4) * EMBED_DIM + c_ids], v)

      pltpu.async_copy(st_v.at[b],
                       out_hbm.at[pl.ds(tc * EMBED_DIM, EMBED_DIM)],
                       osem.at[b])

  for b in range(4):
    drain(osem.at[b], st_v.at[b], out_hbm.at[pl.ds(0, EMBED_DIM)])


@jax.jit
def _retile(wt, wt_tail):
  mesh = plsc.VectorSubcoreMesh(core_axis_name="c", subcore_axis_name="s")
  f = pl.kernel(
      _retile_body,
      out_type=jax.ShapeDtypeStruct((N_TC * EMBED_DIM, UNIT), jnp.float32),
      mesh=mesh,
      scratch_types=[
          pltpu.VMEM((4, EMBED_DIM, UNIT), jnp.float32),
          pltpu.VMEM((4, EMBED_DIM, UNIT), jnp.float32),
          pltpu.SemaphoreType.DMA((4,)),
          pltpu.SemaphoreType.DMA((4,)),
      ],
      compiler_params=pltpu.CompilerParams(
          needs_layout_passes=False, use_tc_tiling_on_sc=True),
  )
  return f(wt, wt_tail)


@jax.jit
def _lookup(idx3d, weight):
  mesh = plsc.VectorSubcoreMesh(core_axis_name="c", subcore_axis_name="s")
  f = pl.kernel(
      _body,
      out_type=jax.ShapeDtypeStruct((SEQ, CBLK, BBLK, 8, UNIT), jnp.float32),
      mesh=mesh,
      scratch_types=[
          pltpu.VMEM((B_PER_W, 8, UNIT), jnp.int32),
          pltpu.VMEM((RING, UNIT, EMBED_DIM), jnp.float32),
          pltpu.VMEM((OUTB, CBLK, 8, UNIT), jnp.float32),
          pltpu.SemaphoreType.DMA((RING,)),
          pltpu.SemaphoreType.DMA((OUTB,)),
      ],
      compiler_params=pltpu.CompilerParams(
          needs_layout_passes=False, use_tc_tiling_on_sc=False),
  )
  return f(weight, idx3d)


def kernel(input_batch, weight):
  wt_tail = jnp.concatenate(
      [weight[NUM_EMBEDDINGS - UNIT // 2:],
       jnp.zeros((UNIT // 2, EMBED_DIM), jnp.float32)]).T
  table = _retile(weight.T, wt_tail).reshape(PAD_ROWS, EMBED_DIM)
  idx3d = (input_batch.T.reshape(SEQ // 8, 8, BBLK, UNIT)
           .transpose(0, 2, 1, 3).reshape(N_BLOCKS, 8, UNIT))
  out5 = _lookup(idx3d, table)
  return out5.transpose(2, 4, 0, 1, 3).reshape(BATCH, SEQ, EMBED_DIM)


# speedup vs baseline: 1.4152x; 1.0010x over previous
import jax
import jax.numpy as jnp
from jax import lax
from jax.experimental import pallas as pl
from jax.experimental.pallas import tpu as pltpu
from jax.experimental.pallas import tpu_sc as plsc

NUM_EMBEDDINGS = 1000000
EMBED_DIM = 32
BATCH = 4096
SEQ = 200
TOTAL = BATCH * SEQ

NC = 2
NS = 16
NW = NC * NS
LANES = 16

UNIT = 128
N_UNITS = TOTAL // UNIT
U_PER_W = N_UNITS // NW
N_BLOCKS = N_UNITS // 8
B_PER_W = N_BLOCKS // NW
CBLK = EMBED_DIM // 8
BBLK = BATCH // UNIT
RING = 6
OUTB = 4


def _body(table_hbm, idx_hbm, out_hbm, idx_v, rows_v, t_v, gsem, osem):
  wid = lax.axis_index("s") * NC + lax.axis_index("c")
  blk0 = wid * B_PER_W

  pltpu.sync_copy(idx_hbm.at[pl.ds(blk0, B_PER_W)], idx_v)

  lane = lax.iota(jnp.int32, LANES)
  def idx_row(k):
    return idx_v.at[k // 8, k % 8]

  def fire_gather(k, r):
    pltpu.async_copy(table_hbm.at[idx_row(k)], rows_v.at[r], gsem.at[r])

  def drain(sem_ref, vmem_ref, hbm_like):
    pltpu.make_async_copy(hbm_like, vmem_ref, sem_ref).wait()

  for r in range(RING - 1):
    fire_gather(r, r)

  @pl.loop(0, U_PER_W)
  def _ring(k):
    r = k % RING
    p = k % OUTB
    b_id = blk0 + k // 8
    s = (b_id // BBLK) * 8 + k % 8
    bt = b_id % BBLK

    @pl.when(k + RING - 1 < U_PER_W)
    def _fire_next():
      fire_gather(k + RING - 1, (k + RING - 1) % RING)

    drain(gsem.at[r], rows_v.at[r], table_hbm.at[pl.ds(0, UNIT)])

    @pl.when(k >= OUTB)
    def _drain_prev_out():
      drain(osem.at[p], t_v.at[p], out_hbm.at[0, :, 0])

    @pl.loop(0, UNIT // LANES)
    def _g(g):
      vidx = idx_row(k)[pl.ds(g * LANES, LANES)]
      keep = vidx != 0
      row_ids = lane + g * LANES
      rot = lane
      for d in range(LANES):
        for h in range(2):
          cols = rot + h * LANES
          v = plsc.load_gather(rows_v.at[r], [row_ids, cols])
          v = jnp.where(keep, v, 0.0)
          plsc.store_scatter(t_v.at[p], [cols // 8, cols % 8, row_ids], v)
        rot = (rot + 1) % LANES

    pltpu.async_copy(t_v.at[p], out_hbm.at[s, :, bt], osem.at[p])

  for p in range(OUTB):
    drain(osem.at[p], t_v.at[p], out_hbm.at[0, :, 0])


N_TC = 7813
PAD_ROWS = N_TC * UNIT


def _retile_body(wt_hbm, tail_hbm, out_hbm, in_v, st_v, isem, osem):
  wid = lax.axis_index("s") * NC + lax.axis_index("c")
  lane = lax.iota(jnp.int32, LANES)
  n_my = N_TC // NW + 1

  def drain(sem_ref, vmem_ref, hbm_like):
    pltpu.make_async_copy(hbm_like, vmem_ref, sem_ref).wait()

  def fire_in(tc, b):
    pltpu.async_copy(wt_hbm.at[:, pl.ds(tc * UNIT, UNIT)], in_v.at[b],
                     isem.at[b])

  for t0 in range(3):
    fire_in(wid + t0 * NW, t0)

  @pl.loop(0, n_my)
  def _unit(t):
    tc = wid + t * NW

    @pl.when(tc < N_TC)
    def _do():
      b = t % 4

      @pl.when(tc + 3 * NW < N_TC - 1)
      def _fire_next():
        fire_in(tc + 3 * NW, (t + 3) % 4)

      @pl.when(tc < N_TC - 1)
      def _drain_in():
        drain(isem.at[b], in_v.at[b], out_hbm.at[pl.ds(0, EMBED_DIM)])

      @pl.when(tc == N_TC - 1)
      def _partial_in():
        pltpu.sync_copy(tail_hbm, in_v.at[b])

      @pl.when(t >= 4)
      def _drain_prev_out():
        drain(osem.at[b], st_v.at[b], out_hbm.at[pl.ds(0, EMBED_DIM)])

      @pl.loop(0, 16)
      def _blk(q):
        lb = q // 2
        ch = q % 2
        j_ids = lane + lb * LANES
        jd4 = j_ids // 4
        jm4 = (j_ids % 4) * EMBED_DIM
        rot = lane + ch * LANES
        for d in range(LANES):
          v = plsc.load_gather(in_v.at[b], [rot, j_ids])
          plsc.store_scatter(st_v.at[b], [jd4, jm4 + rot], v)
          rot = ((rot + 1) % LANES) + ch * LANES

      pltpu.async_copy(st_v.at[b],
                       out_hbm.at[pl.ds(tc * EMBED_DIM, EMBED_DIM)],
                       osem.at[b])

  for b in range(4):
    drain(osem.at[b], st_v.at[b], out_hbm.at[pl.ds(0, EMBED_DIM)])


@jax.jit
def _retile(wt, wt_tail):
  mesh = plsc.VectorSubcoreMesh(core_axis_name="c", subcore_axis_name="s")
  f = pl.kernel(
      _retile_body,
      out_type=jax.ShapeDtypeStruct((N_TC * EMBED_DIM, UNIT), jnp.float32),
      mesh=mesh,
      scratch_types=[
          pltpu.VMEM((4, EMBED_DIM, UNIT), jnp.float32),
          pltpu.VMEM((4, EMBED_DIM, UNIT), jnp.float32),
          pltpu.SemaphoreType.DMA((4,)),
          pltpu.SemaphoreType.DMA((4,)),
      ],
      compiler_params=pltpu.CompilerParams(
          needs_layout_passes=False, use_tc_tiling_on_sc=True),
  )
  return f(wt, wt_tail)


@jax.jit
def _lookup(idx3d, weight):
  mesh = plsc.VectorSubcoreMesh(core_axis_name="c", subcore_axis_name="s")
  f = pl.kernel(
      _body,
      out_type=jax.ShapeDtypeStruct((SEQ, CBLK, BBLK, 8, UNIT), jnp.float32),
      mesh=mesh,
      scratch_types=[
          pltpu.VMEM((B_PER_W, 8, UNIT), jnp.int32),
          pltpu.VMEM((RING, UNIT, EMBED_DIM), jnp.float32),
          pltpu.VMEM((OUTB, CBLK, 8, UNIT), jnp.float32),
          pltpu.SemaphoreType.DMA((RING,)),
          pltpu.SemaphoreType.DMA((OUTB,)),
      ],
      compiler_params=pltpu.CompilerParams(
          needs_layout_passes=False, use_tc_tiling_on_sc=False),
  )
  return f(weight, idx3d)


def kernel(input_batch, weight):
  wt_tail = jnp.concatenate(
      [weight[NUM_EMBEDDINGS - UNIT // 2:],
       jnp.zeros((UNIT // 2, EMBED_DIM), jnp.float32)]).T
  table = _retile(weight.T, wt_tail).reshape(PAD_ROWS, EMBED_DIM)
  idx3d = (input_batch.T.reshape(SEQ // 8, 8, BBLK, UNIT)
           .transpose(0, 2, 1, 3).reshape(N_BLOCKS, 8, UNIT))
  out5 = _lookup(idx3d, table)
  return out5.transpose(2, 4, 0, 1, 3).reshape(BATCH, SEQ, EMBED_DIM)
